# Initial kernel scaffold; baseline (speedup 1.0000x reference)
import jax
import jax.numpy as jnp
from jax.experimental import pallas as pl


def kernel(logits, actions):
    raise NotImplementedError("write your pallas kernel here")


# speedup vs baseline: 2.7754x; 2.7754x over previous
import functools

import jax
import jax.numpy as jnp
from jax import lax
from jax.experimental import pallas as pl
from jax.experimental.pallas import tpu as pltpu

_NEG_INF = float("-inf")
_I32_MAX = 2**31 - 1


def _reduce_body(V, C, NBLK, x_ref, a_ref, lp_ref, mode_ref,
                 m_sc, s_sc, bv_sc, bi_sc, g_sc):
    j = pl.program_id(0)
    B = x_ref.shape[0]
    x = x_ref[...]
    col = j * C + lax.broadcasted_iota(jnp.int32, (B, C), 1)
    x = jnp.where(col < V, x, _NEG_INF)
    bmax = jnp.max(x, axis=-1, keepdims=True)
    bidx = jnp.min(jnp.where(x == bmax, col, _I32_MAX), axis=-1, keepdims=True)
    bsum = jnp.sum(jnp.exp(x - bmax), axis=-1, keepdims=True)
    bg = jnp.sum(jnp.where(col == a_ref[...], x, 0.0), axis=-1, keepdims=True)

    @pl.when(j == 0)
    def _init():
        m_sc[...] = bmax
        s_sc[...] = bsum
        bv_sc[...] = bmax
        bi_sc[...] = bidx
        g_sc[...] = bg

    @pl.when(j > 0)
    def _acc():
        m_old = m_sc[...]
        m_new = jnp.maximum(m_old, bmax)
        s_sc[...] = s_sc[...] * jnp.exp(m_old - m_new) + bsum * jnp.exp(bmax - m_new)
        m_sc[...] = m_new
        better = bmax > bv_sc[...]
        bv_sc[...] = jnp.where(better, bmax, bv_sc[...])
        bi_sc[...] = jnp.where(better, bidx, bi_sc[...])
        g_sc[...] = g_sc[...] + bg

    @pl.when(j == NBLK - 1)
    def _fin():
        lp_ref[...] = g_sc[...] - m_sc[...] - jnp.log(s_sc[...])
        mode_ref[...] = bi_sc[...]


def _fused_pass(logits, actions, C=8192):
    B, V = logits.shape
    NBLK = pl.cdiv(V, C)
    return pl.pallas_call(
        functools.partial(_reduce_body, V, C, NBLK),
        grid=(NBLK,),
        in_specs=[pl.BlockSpec((B, C), lambda j: (0, j)),
                  pl.BlockSpec((B, 1), lambda j: (0, 0))],
        out_specs=[pl.BlockSpec((B, 1), lambda j: (0, 0)),
                   pl.BlockSpec((B, 1), lambda j: (0, 0))],
        out_shape=[jax.ShapeDtypeStruct((B, 1), jnp.float32),
                   jax.ShapeDtypeStruct((B, 1), jnp.int32)],
        scratch_shapes=[pltpu.VMEM((B, 1), jnp.float32),
                        pltpu.VMEM((B, 1), jnp.float32),
                        pltpu.VMEM((B, 1), jnp.float32),
                        pltpu.VMEM((B, 1), jnp.int32),
                        pltpu.VMEM((B, 1), jnp.float32)],
    )(logits, actions)


def kernel(logits, actions):
    a = actions.astype(jnp.int32)
    log_probs, mode = _fused_pass(logits, a)
    return log_probs, mode
